# Initial kernel scaffold; baseline (speedup 1.0000x reference)
import jax
import jax.numpy as jnp
from jax.experimental import pallas as pl


def kernel(mask, table):
    raise NotImplementedError("write your pallas kernel here")


# speedup vs baseline: 1.8427x; 1.8427x over previous
import functools

import jax
import jax.numpy as jnp
from jax import lax
from jax.experimental import pallas as pl
from jax.experimental.pallas import tpu as pltpu
from jax.experimental.pallas import tpu_sc as plsc

NC = 2
NS = 16
NW = NC * NS

CHUNK = 1024


def _make_gather(n, d):
    assert n % NW == 0
    per_w = n // NW
    assert per_w % CHUNK == 0
    n_chunks = per_w // CHUNK
    mesh = plsc.VectorSubcoreMesh(core_axis_name="c", subcore_axis_name="s")

    @functools.partial(
        pl.kernel,
        out_type=jax.ShapeDtypeStruct((n, d), jnp.float32),
        mesh=mesh,
        scratch_types=[
            pltpu.VMEM((CHUNK,), jnp.int32),
            pltpu.VMEM((CHUNK, d), jnp.float32),
            pltpu.SemaphoreType.DMA,
        ],
        compiler_params=pltpu.CompilerParams(use_tc_tiling_on_sc=False),
    )
    def gather_kernel(table_hbm, idx_hbm, out_hbm, idx_v, rows_v, sem):
        wid = lax.axis_index("s") * NC + lax.axis_index("c")
        base = wid * per_w

        def body(i, _):
            off = base + i * CHUNK
            pltpu.sync_copy(idx_hbm.at[pl.ds(off, CHUNK)], idx_v)
            pltpu.async_copy(table_hbm.at[idx_v], rows_v, sem).wait()
            pltpu.sync_copy(rows_v, out_hbm.at[pl.ds(off, CHUNK)])
            return 0

        lax.fori_loop(0, n_chunks, body, 0)

    return gather_kernel


def kernel(mask, table):
    b, h = mask.shape
    v, d = table.shape
    n = b * h
    idx = mask.reshape(n).astype(jnp.int32)
    out = _make_gather(n, d)(table, idx)
    return out.reshape(b, h, d)


# speedup vs baseline: 1.8751x; 1.0176x over previous
import functools

import jax
import jax.numpy as jnp
from jax import lax
from jax.experimental import pallas as pl
from jax.experimental.pallas import tpu as pltpu
from jax.experimental.pallas import tpu_sc as plsc

NC = 2
NS = 16
NW = NC * NS

CHUNK = 640


def _make_gather(n, d):
    assert n % NW == 0
    per_w = n // NW
    assert per_w % CHUNK == 0
    n_chunks = per_w // CHUNK
    assert n_chunks % 2 == 0 and n_chunks >= 4
    mesh = plsc.VectorSubcoreMesh(core_axis_name="c", subcore_axis_name="s")

    @functools.partial(
        pl.kernel,
        out_type=jax.ShapeDtypeStruct((n, d), jnp.float32),
        mesh=mesh,
        scratch_types=[
            pltpu.VMEM((per_w,), jnp.int32),
            pltpu.VMEM((CHUNK, d), jnp.float32),
            pltpu.VMEM((CHUNK, d), jnp.float32),
            pltpu.SemaphoreType.DMA,
            pltpu.SemaphoreType.DMA,
            pltpu.SemaphoreType.DMA,
            pltpu.SemaphoreType.DMA,
        ],
        compiler_params=pltpu.CompilerParams(use_tc_tiling_on_sc=False),
    )
    def gather_kernel(table_hbm, idx_hbm, out_hbm, idx_v, rows0, rows1,
                      g0, g1, w0, w1):
        wid = lax.axis_index("s") * NC + lax.axis_index("c")
        base = wid * per_w
        rows = (rows0, rows1)
        gsem = (g0, g1)
        wsem = (w0, w1)

        def start_gather(i, b):
            pltpu.async_copy(
                table_hbm.at[idx_v.at[pl.ds(i * CHUNK, CHUNK)]],
                rows[b], gsem[b])

        def wait_gather(b):
            pltpu.make_async_copy(
                table_hbm.at[idx_v.at[pl.ds(0, CHUNK)]],
                rows[b], gsem[b]).wait()

        def start_write(i, b):
            pltpu.async_copy(
                rows[b], out_hbm.at[pl.ds(base + i * CHUNK, CHUNK)], wsem[b])

        def wait_write(b):
            pltpu.make_async_copy(
                rows[b], out_hbm.at[pl.ds(base, CHUNK)], wsem[b]).wait()

        pltpu.sync_copy(idx_hbm.at[pl.ds(base, per_w)], idx_v)

        start_gather(0, 0)
        wait_gather(0)
        start_gather(1, 1)
        start_write(0, 0)

        def steady(k, _):
            def one(i, b):
                wait_gather(b)
                wait_write(1 - b)
                start_gather(i + 1, 1 - b)
                start_write(i, b)
            one(1 + 2 * k, 1)
            one(2 + 2 * k, 0)
            return 0

        lax.fori_loop(0, (n_chunks - 2) // 2, steady, 0)

        wait_gather(1)
        start_write(n_chunks - 1, 1)
        wait_write(0)
        wait_write(1)

    return gather_kernel


def kernel(mask, table):
    b, h = mask.shape
    v, d = table.shape
    n = b * h
    idx = mask.reshape(n).astype(jnp.int32)
    out = _make_gather(n, d)(table, idx)
    return out.reshape(b, h, d)
